# baseline (speedup 1.0000x reference)
import functools

import jax
import jax.numpy as jnp
from jax import lax
from jax.experimental import pallas as pl
from jax.experimental.pallas import tpu as pltpu
from jax.experimental.pallas import tpu_sc as plsc

B, N, D = 128, 32768, 16
NC, NS, L = 2, 16, 16
NW = NC * NS
RPW = B // NW
CHUNKS = N // L
S = 4
UNROLL = 4

_IBIG = jnp.int32(0x7FFFFFFF)


def _scan_row(row_ref, sres):
    lanes = lax.iota(jnp.int32, L)
    neg_inf = jnp.full((L,), -jnp.inf, jnp.float32)

    def vbody(i, c):
        out = []
        for u in range(S):
            m1, m2 = c[u]
            v = row_ref[pl.ds((i * S + u) * L, L)]
            t = jnp.minimum(m1, v)
            out.append((jnp.maximum(m1, v), jnp.maximum(m2, t)))
        return tuple(out)

    vals = plsc.parallel_loop(0, CHUNKS // S, 1, unroll=UNROLL,
                              carry=tuple((neg_inf, neg_inf)
                                          for _ in range(S)))(vbody)

    m1L, m2L = vals[0]
    for m1b, m2b in vals[1:]:
        m2L = jnp.maximum(jnp.minimum(m1L, m1b), jnp.maximum(m2L, m2b))
        m1L = jnp.maximum(m1L, m1b)
    M1 = jnp.max(m1L)
    lmin = jnp.min(jnp.where(m1L == M1, lanes, jnp.int32(L)))
    ca = jnp.where((m1L == M1) & (lanes == lmin), neg_inf, m1L)
    M2 = jnp.maximum(jnp.max(m2L), jnp.max(ca))

    cbig = jnp.full((L,), CHUNKS, jnp.int32)

    @pl.when(M2 < M1)
    def _():
        def ibody(i, c):
            out = []
            for u in range(S):
                ci = i * S + u
                v = row_ref[pl.ds(ci * L, L)]
                out.append(jnp.minimum(c[u],
                                       jnp.where(v == M2, ci, CHUNKS)))
            return tuple(out)
        accs = plsc.parallel_loop(0, CHUNKS // S, 1, unroll=UNROLL,
                                  carry=(cbig,) * S)(ibody)
        idxs = [a * L + lanes for a in accs]
        sres[0] = jnp.min(functools.reduce(jnp.minimum, idxs))

    @pl.when(M2 == M1)
    def _():
        def dbody(i, c):
            out = []
            for u in range(S):
                a, b = c[u]
                ci = i * S + u
                v = row_ref[pl.ds(ci * L, L)]
                cand = jnp.where(v == M1, ci, CHUNKS)
                out.append((jnp.minimum(a, cand),
                            jnp.minimum(b, jnp.maximum(a, cand))))
            return tuple(out)
        accs = plsc.parallel_loop(0, CHUNKS // S, 1, unroll=UNROLL,
                                  carry=((cbig, cbig),) * S)(dbody)
        ias = [a * L + lanes for a, _ in accs]
        ibs = [b * L + lanes for _, b in accs]
        g1 = jnp.min(functools.reduce(jnp.minimum, ias))
        seconds = [jnp.where(ia == g1, _IBIG, ia) for ia in ias]
        sres[0] = jnp.minimum(
            jnp.min(functools.reduce(jnp.minimum, seconds)),
            jnp.min(functools.reduce(jnp.minimum, ibs)))

    return sres[0]


def _sc_body(probs_hbm, poses_t_hbm, out_hbm, row_a, row_b, win_v, pose_v,
             sres, sem_a, sem_b, sem_p):
    wid = lax.axis_index("s") * NC + lax.axis_index("c")
    base = wid * RPW
    bufs = ((row_a, sem_a), (row_b, sem_b))
    lanes = lax.iota(jnp.int32, L)

    pltpu.async_copy(probs_hbm.at[base], row_a, sem_a)
    sels = []
    for r in range(RPW):
        row_ref, sem = bufs[r % 2]
        pltpu.make_async_copy(probs_hbm.at[base + r], row_ref, sem).wait()
        if r + 1 < RPW:
            nref, nsem = bufs[(r + 1) % 2]
            pltpu.async_copy(probs_hbm.at[base + r + 1], nref, nsem)
        sel = _scan_row(row_ref, sres)
        col0 = pl.multiple_of((sel // 128) * 128, 128)
        pltpu.async_copy(poses_t_hbm.at[base + r, :, pl.ds(col0, 128)],
                         win_v.at[r], sem_p)
        sels.append((sel, col0))
    for r, (sel, col0) in enumerate(sels):
        pltpu.make_async_copy(
            poses_t_hbm.at[base + r, :, pl.ds(col0, 128)],
            win_v.at[r], sem_p).wait()
        col = jnp.full((L,), sel - col0, jnp.int32)
        pose_v[r, :] = plsc.load_gather(win_v.at[r], [lanes, col])
    pltpu.sync_copy(pose_v, out_hbm.at[pl.ds(base, RPW)])


@jax.jit
def kernel(poses, probs, labels):
    del labels
    poses_t = jnp.transpose(poses, (0, 2, 1))
    mesh = plsc.VectorSubcoreMesh(core_axis_name="c", subcore_axis_name="s",
                                  num_cores=NC, num_subcores=NS)
    run = pl.kernel(
        _sc_body,
        out_type=jax.ShapeDtypeStruct((B, D), jnp.float32),
        mesh=mesh,
        compiler_params=pltpu.CompilerParams(needs_layout_passes=False),
        scratch_types=[
            pltpu.VMEM((N,), jnp.float32),
            pltpu.VMEM((N,), jnp.float32),
            pltpu.VMEM((RPW, D, 128), jnp.float32),
            pltpu.VMEM((RPW, D), jnp.float32),
            pltpu.SMEM((1,), jnp.int32),
            pltpu.SemaphoreType.DMA,
            pltpu.SemaphoreType.DMA,
            pltpu.SemaphoreType.DMA,
        ],
    )
    return run(probs, poses_t)


# speedup vs baseline: 1.0260x; 1.0260x over previous
import functools

import jax
import jax.numpy as jnp
from jax import lax
from jax.experimental import pallas as pl
from jax.experimental.pallas import tpu as pltpu
from jax.experimental.pallas import tpu_sc as plsc

B, N, D = 128, 32768, 16
NC, NS, L = 2, 16, 16
NW = NC * NS
RPW = B // NW
CHUNKS = N // L
S = 4
UNROLL = 4

_IBIG = jnp.int32(0x7FFFFFFF)


def _scan_row(row_ref):
    lanes = lax.iota(jnp.int32, L)
    neg_inf = jnp.full((L,), -jnp.inf, jnp.float32)
    zeros_i = jnp.zeros((L,), jnp.int32)

    init = tuple((neg_inf, zeros_i, neg_inf, zeros_i) for _ in range(S))

    def body(i, c):
        out = []
        for u in range(S):
            m1, c1, m2, c2 = c[u]
            ci = i * S + u
            v = row_ref[pl.ds(ci * L, L)]
            gt1 = v > m1
            gt2 = v > m2
            m2n = jnp.where(gt1, m1, jnp.where(gt2, v, m2))
            c2n = jnp.where(gt1, c1, jnp.where(gt2, ci, c2))
            m1n = jnp.where(gt1, v, m1)
            c1n = jnp.where(gt1, ci, c1)
            out.append((m1n, c1n, m2n, c2n))
        return tuple(out)

    states = plsc.parallel_loop(0, CHUNKS // S, 1, unroll=UNROLL,
                                carry=init)(body)

    m1s = [s[0] for s in states]
    i1s = [s[1] * L + lanes for s in states]
    m2s = [s[2] for s in states]
    i2s = [s[3] * L + lanes for s in states]

    M1 = jnp.max(functools.reduce(jnp.maximum, m1s))
    i1g = functools.reduce(
        jnp.minimum,
        [jnp.min(jnp.where(m1 == M1, i1, _IBIG))
         for m1, i1 in zip(m1s, i1s)])
    cas = [jnp.where((m1 == M1) & (i1 == i1g), neg_inf, m1)
           for m1, i1 in zip(m1s, i1s)]
    M2 = jnp.maximum(jnp.max(functools.reduce(jnp.maximum, cas)),
                     jnp.max(functools.reduce(jnp.maximum, m2s)))
    sel = jnp.minimum(
        functools.reduce(
            jnp.minimum,
            [jnp.min(jnp.where(ca == M2, i1, _IBIG))
             for ca, i1 in zip(cas, i1s)]),
        functools.reduce(
            jnp.minimum,
            [jnp.min(jnp.where(m2 == M2, i2, _IBIG))
             for m2, i2 in zip(m2s, i2s)]))
    return sel


def _sc_body(probs_hbm, poses_t_hbm, out_hbm, row_a, row_b, win_v, pose_v,
             sem_a, sem_b, sem_p):
    wid = lax.axis_index("s") * NC + lax.axis_index("c")
    base = wid * RPW
    bufs = ((row_a, sem_a), (row_b, sem_b))
    lanes = lax.iota(jnp.int32, L)

    pltpu.async_copy(probs_hbm.at[base], row_a, sem_a)
    sels = []
    for r in range(RPW):
        row_ref, sem = bufs[r % 2]
        pltpu.make_async_copy(probs_hbm.at[base + r], row_ref, sem).wait()
        if r + 1 < RPW:
            nref, nsem = bufs[(r + 1) % 2]
            pltpu.async_copy(probs_hbm.at[base + r + 1], nref, nsem)
        sel = _scan_row(row_ref)
        col0 = pl.multiple_of((sel // 128) * 128, 128)
        pltpu.async_copy(poses_t_hbm.at[base + r, :, pl.ds(col0, 128)],
                         win_v.at[r], sem_p)
        sels.append((sel, col0))
    for r, (sel, col0) in enumerate(sels):
        pltpu.make_async_copy(
            poses_t_hbm.at[base + r, :, pl.ds(col0, 128)],
            win_v.at[r], sem_p).wait()
        col = jnp.full((L,), sel - col0, jnp.int32)
        pose_v[r, :] = plsc.load_gather(win_v.at[r], [lanes, col])
    pltpu.sync_copy(pose_v, out_hbm.at[pl.ds(base, RPW)])


@jax.jit
def kernel(poses, probs, labels):
    del labels
    poses_t = jnp.transpose(poses, (0, 2, 1))
    mesh = plsc.VectorSubcoreMesh(core_axis_name="c", subcore_axis_name="s",
                                  num_cores=NC, num_subcores=NS)
    run = pl.kernel(
        _sc_body,
        out_type=jax.ShapeDtypeStruct((B, D), jnp.float32),
        mesh=mesh,
        compiler_params=pltpu.CompilerParams(needs_layout_passes=False),
        scratch_types=[
            pltpu.VMEM((N,), jnp.float32),
            pltpu.VMEM((N,), jnp.float32),
            pltpu.VMEM((RPW, D, 128), jnp.float32),
            pltpu.VMEM((RPW, D), jnp.float32),
            pltpu.SemaphoreType.DMA,
            pltpu.SemaphoreType.DMA,
            pltpu.SemaphoreType.DMA,
        ],
    )
    return run(probs, poses_t)
